# baseline (device time: 20914 ns/iter reference)
import jax
import jax.numpy as jnp
from jax import lax
from jax.experimental import pallas as pl
from jax.experimental.pallas import tpu as pltpu

N_DEV = 4
N_HOPS = N_DEV - 1
SEED = N_HOPS


def kernel(x):
    m_per, n = x.shape
    half = m_per // 2

    def body(x_ref, out_ref, xv, comm_ref, in_sems, out_sems,
             send_sems, recv_sems):
        my_pos = lax.axis_index("i")
        left = lax.rem(my_pos + N_DEV - 1, N_DEV)
        right = lax.rem(my_pos + 1, N_DEV)

        cin = [
            pltpu.make_async_copy(
                x_ref.at[pl.ds(i * half, half)],
                xv.at[pl.ds(i * half, half)],
                in_sems.at[i],
            )
            for i in range(2)
        ]
        cin[0].start()
        cin[1].start()

        barrier_sem = pltpu.get_barrier_semaphore()
        for nbr in (left, right):
            pl.semaphore_signal(
                barrier_sem, inc=1,
                device_id=(nbr,), device_id_type=pl.DeviceIdType.MESH,
            )
        pl.semaphore_wait(barrier_sem, 2)

        def hop(h):
            src = SEED if h == 0 else h - 1
            return pltpu.make_async_remote_copy(
                src_ref=comm_ref.at[src],
                dst_ref=comm_ref.at[h],
                send_sem=send_sems.at[h],
                recv_sem=recv_sems.at[h],
                device_id=(right,),
                device_id_type=pl.DeviceIdType.MESH,
            )

        with jax.named_scope("seed"):
            cin[0].wait()
            ta = xv[pl.ds(0, half), :]
            rows = half
            while rows > 1:
                r2 = rows // 2
                ta = ta[:r2, :] * ta[r2:rows, :]
                rows = r2
            cin[1].wait()
            tb = xv[pl.ds(half, half), :]
            rows = half
            while rows > 1:
                r2 = rows // 2
                tb = tb[:r2, :] * tb[r2:rows, :]
                rows = r2
            comm_ref[SEED, :, :] = ta * tb
            r0 = hop(0)
            r0.start()

        with jax.named_scope("phaseA"):
            y = xv[:, :]
            for shift in (1, 2, 4, 8, 16, 32):
                pad = jnp.ones((shift, n), dtype=y.dtype)
                y = y * jnp.concatenate([pad, y[: m_per - shift, :]], axis=0)

        with jax.named_scope("wait0"):
            r0.wait_recv()
            r1 = hop(1)
            r1.start()

        with jax.named_scope("phaseB"):
            for shift in (64, 128, 256):
                pad = jnp.ones((shift, n), dtype=y.dtype)
                y = y * jnp.concatenate([pad, y[: m_per - shift, :]], axis=0)

        with jax.named_scope("wait1"):
            r1.wait_recv()
            r2 = hop(2)
            r2.start()

        with jax.named_scope("phaseC"):
            for shift in (512, 1024):
                pad = jnp.ones((shift, n), dtype=y.dtype)
                y = y * jnp.concatenate([pad, y[: m_per - shift, :]], axis=0)

        with jax.named_scope("wait2"):
            r2.wait_recv()

        with jax.named_scope("final"):
            prefix = jnp.ones((1, n), dtype=y.dtype)
            for h in range(N_HOPS):
                v = comm_ref[h, :, :]
                prefix = prefix * jnp.where(h < my_pos, v, jnp.ones_like(v))

            couts = []
            for i in range(2):
                xv[pl.ds(i * half, half), :] = (
                    y[i * half : (i + 1) * half, :] * prefix
                )
                cp = pltpu.make_async_copy(
                    xv.at[pl.ds(i * half, half)],
                    out_ref.at[pl.ds(i * half, half)],
                    out_sems.at[i],
                )
                cp.start()
                couts.append(cp)

            couts[0].wait()
            couts[1].wait()

            r0.wait_send()
            r1.wait_send()
            r2.wait_send()

    return pl.pallas_call(
        body,
        out_shape=jax.ShapeDtypeStruct((m_per, n), x.dtype),
        in_specs=[pl.BlockSpec(memory_space=pltpu.MemorySpace.HBM)],
        out_specs=pl.BlockSpec(memory_space=pltpu.MemorySpace.HBM),
        scratch_shapes=[
            pltpu.VMEM((m_per, n), x.dtype),
            pltpu.VMEM((N_HOPS + 1, 1, n), x.dtype),
            pltpu.SemaphoreType.DMA((2,)),
            pltpu.SemaphoreType.DMA((2,)),
            pltpu.SemaphoreType.DMA((N_HOPS,)),
            pltpu.SemaphoreType.DMA((N_HOPS,)),
        ],
        compiler_params=pltpu.CompilerParams(collective_id=0),
    )(x)


# device time: 18900 ns/iter; 1.1066x vs baseline; 1.1066x over previous
import jax
import jax.numpy as jnp
from jax import lax
from jax.experimental import pallas as pl
from jax.experimental.pallas import tpu as pltpu

N_DEV = 4
SEED = 0
SLOT_L = 1
SLOT_U = 2
SLOT_W = 3


def kernel(x):
    m_per, n = x.shape
    half = m_per // 2

    def body(x_ref, out_ref, comm_ref, send_sems, recv_sems):
        my_pos = lax.axis_index("i")
        left = lax.rem(my_pos + N_DEV - 1, N_DEV)
        right = lax.rem(my_pos + 1, N_DEV)
        across = lax.rem(my_pos + 2, N_DEV)

        barrier_sem = pltpu.get_barrier_semaphore()
        for nbr in (left, right):
            pl.semaphore_signal(
                barrier_sem, inc=1,
                device_id=(nbr,), device_id_type=pl.DeviceIdType.MESH,
            )
        pl.semaphore_wait(barrier_sem, 2)

        with jax.named_scope("seed"):
            t = x_ref[:, :]
            rows = m_per
            while rows > 1:
                r2 = rows // 2
                t = t[:r2, :] * t[r2:rows, :]
                rows = r2
            comm_ref[SEED, :, :] = t
            step0 = pltpu.make_async_remote_copy(
                src_ref=comm_ref.at[SEED],
                dst_ref=comm_ref.at[SLOT_L],
                send_sem=send_sems.at[0],
                recv_sem=recv_sems.at[0],
                device_id=(right,),
                device_id_type=pl.DeviceIdType.MESH,
            )
            step0.start()

        with jax.named_scope("phaseA"):
            y = x_ref[:, :]
            for shift in (1, 2, 4):
                pad = jnp.ones((shift, n), dtype=y.dtype)
                y = y * jnp.concatenate([pad, y[: m_per - shift, :]], axis=0)

        with jax.named_scope("wait0"):
            step0.wait_recv()
            l = comm_ref[SLOT_L, :, :]
            ones_row = jnp.ones((1, n), dtype=y.dtype)
            comm_ref[SLOT_U, :, :] = jnp.where(my_pos >= 1, l, ones_row) * t
            step1 = pltpu.make_async_remote_copy(
                src_ref=comm_ref.at[SLOT_U],
                dst_ref=comm_ref.at[SLOT_W],
                send_sem=send_sems.at[1],
                recv_sem=recv_sems.at[1],
                device_id=(across,),
                device_id_type=pl.DeviceIdType.MESH,
            )
            step1.start()

        with jax.named_scope("phaseB"):
            for shift in (8, 16, 32, 64, 128, 256, 512):
                pad = jnp.ones((shift, n), dtype=y.dtype)
                y = y * jnp.concatenate([pad, y[: m_per - shift, :]], axis=0)

        with jax.named_scope("wait1"):
            step1.wait_recv()

        with jax.named_scope("final"):
            w = comm_ref[SLOT_W, :, :]
            prefix = jnp.where(my_pos >= 1, l, ones_row) * jnp.where(
                my_pos >= 2, w, ones_row
            )
            tp = y[:half, :] * prefix
            out_ref[:half, :] = tp
            out_ref[half:, :] = y[half:, :] * tp

            step0.wait_send()
            step1.wait_send()

    return pl.pallas_call(
        body,
        out_shape=jax.ShapeDtypeStruct((m_per, n), x.dtype),
        in_specs=[pl.BlockSpec(memory_space=pltpu.VMEM)],
        out_specs=pl.BlockSpec(memory_space=pltpu.VMEM),
        scratch_shapes=[
            pltpu.VMEM((4, 1, n), x.dtype),
            pltpu.SemaphoreType.DMA((2,)),
            pltpu.SemaphoreType.DMA((2,)),
        ],
        compiler_params=pltpu.CompilerParams(collective_id=0),
    )(x)


# device time: 17970 ns/iter; 1.1638x vs baseline; 1.0518x over previous
import jax
import jax.numpy as jnp
from jax import lax
from jax.experimental import pallas as pl
from jax.experimental.pallas import tpu as pltpu

N_DEV = 4
SEED = 0
SLOT_L = 1
SLOT_U = 2
SLOT_W = 3


def kernel(x):
    m_per, n = x.shape
    half = m_per // 2

    def body(x_ref, out_ref, comm_ref, send_sems, recv_sems):
        my_pos = lax.axis_index("i")
        left = lax.rem(my_pos + N_DEV - 1, N_DEV)
        right = lax.rem(my_pos + 1, N_DEV)
        across = lax.rem(my_pos + 2, N_DEV)

        barrier_sem = pltpu.get_barrier_semaphore()
        for nbr in (left, right):
            pl.semaphore_signal(
                barrier_sem, inc=1,
                device_id=(nbr,), device_id_type=pl.DeviceIdType.MESH,
            )
        pl.semaphore_wait(barrier_sem, 2)

        with jax.named_scope("seed"):
            t = x_ref[:, :]
            rows = m_per
            while rows > 1:
                r2 = rows // 2
                t = t[:r2, :] * t[r2:rows, :]
                rows = r2
            comm_ref[SEED, :, :] = t
            step0 = pltpu.make_async_remote_copy(
                src_ref=comm_ref.at[SEED],
                dst_ref=comm_ref.at[SLOT_L],
                send_sem=send_sems.at[0],
                recv_sem=recv_sems.at[0],
                device_id=(right,),
                device_id_type=pl.DeviceIdType.MESH,
            )
            step0.start()

        with jax.named_scope("phaseA"):
            y = x_ref[:, :]
            for shift in (1, 2, 4):
                pad = jnp.ones((shift, n), dtype=y.dtype)
                y = y * jnp.concatenate([pad, y[: m_per - shift, :]], axis=0)

        with jax.named_scope("wait0"):
            step0.wait_recv()
            l = comm_ref[SLOT_L, :, :]
            ones_row = jnp.ones((1, n), dtype=y.dtype)
            comm_ref[SLOT_U, :, :] = jnp.where(my_pos >= 1, l, ones_row) * t
            step1 = pltpu.make_async_remote_copy(
                src_ref=comm_ref.at[SLOT_U],
                dst_ref=comm_ref.at[SLOT_W],
                send_sem=send_sems.at[1],
                recv_sem=recv_sems.at[1],
                device_id=(across,),
                device_id_type=pl.DeviceIdType.MESH,
            )
            step1.start()

        with jax.named_scope("phaseB"):
            for shift in (8, 16, 32, 64, 128):
                pad = jnp.ones((shift, n), dtype=y.dtype)
                y = y * jnp.concatenate([pad, y[: m_per - shift, :]], axis=0)

        with jax.named_scope("wait1"):
            step1.wait_recv()

        with jax.named_scope("final"):
            w = comm_ref[SLOT_W, :, :]
            prefix = jnp.where(my_pos >= 1, l, ones_row) * jnp.where(
                my_pos >= 2, w, ones_row
            )
            B = 256
            blk = y[:B, :] * prefix
            out_ref[:B, :] = blk
            for b in range(1, m_per // B):
                blk = y[b * B : (b + 1) * B, :] * blk
                out_ref[b * B : (b + 1) * B, :] = blk

            step0.wait_send()
            step1.wait_send()

    return pl.pallas_call(
        body,
        out_shape=jax.ShapeDtypeStruct((m_per, n), x.dtype),
        in_specs=[pl.BlockSpec(memory_space=pltpu.VMEM)],
        out_specs=pl.BlockSpec(memory_space=pltpu.VMEM),
        scratch_shapes=[
            pltpu.VMEM((4, 1, n), x.dtype),
            pltpu.SemaphoreType.DMA((2,)),
            pltpu.SemaphoreType.DMA((2,)),
        ],
        compiler_params=pltpu.CompilerParams(collective_id=0),
    )(x)
